# baseline (speedup 1.0000x reference)
import functools
import math

import jax
import jax.numpy as jnp
from jax import lax
from jax.experimental import pallas as pl
from jax.experimental.pallas import tpu as pltpu
from jax.experimental.pallas import tpu_sc as plsc

D = 1024
D2 = 512
NH = 8
HD = D // NH
BT = 256


def _gelu(x):
    return 0.5 * x * (1.0 + lax.erf(x * (1.0 / math.sqrt(2.0))))


def _dot(a, b):
    return jnp.dot(a, b, preferred_element_type=jnp.float32)


_SC_CHUNK = 32


def _sc_gather_rows(table, idx):
    n_rows = idx.shape[0]
    info = plsc.get_sparse_core_info()
    num_workers = info.num_cores * info.num_subcores
    rows_per_w = n_rows // num_workers
    chunk = _SC_CHUNK
    n_chunks = rows_per_w // chunk
    mesh = plsc.VectorSubcoreMesh(core_axis_name="c", subcore_axis_name="s")

    @functools.partial(
        pl.kernel,
        mesh=mesh,
        out_type=jax.ShapeDtypeStruct((n_rows, D), jnp.float32),
        scratch_types=[
            pltpu.VMEM((2, chunk), jnp.int32),
            pltpu.VMEM((2, chunk, D), jnp.float32),
            pltpu.SemaphoreType.DMA,
            pltpu.SemaphoreType.DMA,
        ],
    )
    def k(table_hbm, idx_hbm, out_hbm, idx_v, rows_v, gsem, ssem):
        wid = lax.axis_index("s") * info.num_cores + lax.axis_index("c")
        base = wid * rows_per_w
        gh = [None] * n_chunks
        sh = [None] * n_chunks
        for c in range(n_chunks):
            buf = c % 2
            if c >= 2:
                sh[c - 2].wait()
            pltpu.sync_copy(idx_hbm.at[pl.ds(base + c * chunk, chunk)],
                            idx_v.at[buf])
            gh[c] = pltpu.async_copy(table_hbm.at[idx_v.at[buf]],
                                     rows_v.at[buf], gsem)
            if c >= 1:
                gh[c - 1].wait()
                sh[c - 1] = pltpu.async_copy(
                    rows_v.at[1 - buf],
                    out_hbm.at[pl.ds(base + (c - 1) * chunk, chunk)], ssem)
        gh[n_chunks - 1].wait()
        sh[n_chunks - 1] = pltpu.async_copy(
            rows_v.at[(n_chunks - 1) % 2],
            out_hbm.at[pl.ds(base + (n_chunks - 1) * chunk, chunk)], ssem)
        sh[n_chunks - 2].wait()
        sh[n_chunks - 1].wait()

    return k(table, idx)


def _sc_scatter_rows(table, idx):
    n_rows = idx.shape[0]
    info = plsc.get_sparse_core_info()
    num_workers = info.num_cores * info.num_subcores
    rows_per_w = n_rows // num_workers
    chunk = _SC_CHUNK
    n_chunks = rows_per_w // chunk
    mesh = plsc.VectorSubcoreMesh(core_axis_name="c", subcore_axis_name="s")

    @functools.partial(
        pl.kernel,
        mesh=mesh,
        out_type=jax.ShapeDtypeStruct((n_rows, D), jnp.float32),
        scratch_types=[
            pltpu.VMEM((2, chunk), jnp.int32),
            pltpu.VMEM((2, chunk, D), jnp.float32),
            pltpu.SemaphoreType.DMA,
        ],
    )
    def k(table_hbm, idx_hbm, out_hbm, idx_v, rows_v, wsem):
        wid = lax.axis_index("s") * info.num_cores + lax.axis_index("c")
        base = wid * rows_per_w
        wh = [None] * n_chunks
        for c in range(n_chunks):
            buf = c % 2
            if c >= 2:
                wh[c - 2].wait()
            off = base + c * chunk
            pltpu.sync_copy(idx_hbm.at[pl.ds(off, chunk)], idx_v.at[buf])
            pltpu.sync_copy(table_hbm.at[pl.ds(off, chunk)], rows_v.at[buf])
            wh[c] = pltpu.async_copy(rows_v.at[buf], out_hbm.at[idx_v.at[buf]],
                                     wsem)
        wh[n_chunks - 2].wait()
        wh[n_chunks - 1].wait()

    return k(table, idx)


def _k_experts012(offs_ref, x_ref,
                  mw1, mb1, mw2, mb2,
                  sw1, sb1, sw2, sb2,
                  aw1, ab1, aw2, ab2,
                  bw1, bb1, bw2, bb2, bw3, bb3,
                  cw1a, cw1b, cb1, cw2, cb2,
                  gw1m, gw1t, gb1, gw2, gb2,
                  o_ref):
    b = pl.program_id(0)
    j = pl.program_id(1)
    bt = x_ref.shape[1]
    row0 = j * bt
    o1 = offs_ref[b, 0]
    o2 = offs_ref[b, 1]
    o3 = offs_ref[b, 2]
    rid = row0 + lax.broadcasted_iota(jnp.int32, (bt, 1), 0)
    x = x_ref[0]

    @pl.when(row0 < o1)
    def _():
        h = jnp.maximum(_dot(x, mw1[...]) + mb1[...], 0.0)
        e0 = _dot(h, mw2[...]) + mb2[...]
        o_ref[0] = jnp.where(rid < o1, e0, o_ref[0])

    @pl.when((row0 < o2) & (row0 + bt > o1))
    def _():
        h = _gelu(_dot(x, sw1[...]) + sb1[...])
        e1 = _dot(h, sw2[...]) + sb2[...]
        o_ref[0] = jnp.where((rid >= o1) & (rid < o2), e1, o_ref[0])

    @pl.when((row0 < o3) & (row0 + bt > o2))
    def _():
        a = _dot(_gelu(_dot(x, aw1[...]) + ab1[...]), aw2[...]) + ab2[...]
        t = _gelu(_dot(x, bw1[...]) + bb1[...])
        t = _gelu(_dot(t, bw2[...]) + bb2[...])
        bb = _dot(t, bw3[...]) + bb3[...]
        h = _gelu(_dot(a, cw1a[...]) + _dot(bb, cw1b[...]) + cb1[...])
        e2 = _dot(h, cw2[...]) + cb2[...]
        o_ref[0] = jnp.where((rid >= o2) & (rid < o3), e2, o_ref[0])

    @pl.when(row0 < o3)
    def _():
        pr = o_ref[0]
        tier = ((rid >= o1).astype(jnp.int32) + (rid >= o2).astype(jnp.int32)
                + (rid >= o3).astype(jnp.int32))
        cols = lax.broadcasted_iota(jnp.int32, (bt, 8), 1)
        onehot = (tier == cols).astype(jnp.float32)
        h = _gelu(_dot(pr, gw1m[...]) + _dot(onehot, gw1t[...]) + gb1[...])
        logit = _dot(h, gw2[...]) + gb2[...]
        o_ref[0] = pr * jax.nn.sigmoid(logit[:, 0:1])


def _k_ffn_res2(offs_ref, x_ref, w1a, b1a, w2a, b2a, w1b, b1b, w2b, b2b, o_ref):
    b = pl.program_id(0)

    @pl.when(offs_ref[b, 3] > offs_ref[b, 2])
    def _():
        x = x_ref[0]
        h = _gelu(_dot(x, w1a[...]) + b1a[...])
        x = x + _dot(h, w2a[...]) + b2a[...]
        h = _gelu(_dot(x, w1b[...]) + b1b[...])
        o_ref[0] = x + _dot(h, w2b[...]) + b2b[...]


def _k_ffn_res_kv(offs_ref, x_ref, w1, b1, w2, b2, wk, bk, wv, bv,
                  o_ref, k_ref, v_ref):
    b = pl.program_id(0)

    @pl.when(offs_ref[b, 3] > offs_ref[b, 2])
    def _():
        x = x_ref[0]
        h = _gelu(_dot(x, w1[...]) + b1[...])
        x3 = x + _dot(h, w2[...]) + b2[...]
        o_ref[0] = x3
        k_ref[0] = (_dot(x3, wk[...]) + bk[...]).astype(jnp.bfloat16)
        v_ref[0] = (_dot(x3, wv[...]) + bv[...]).astype(jnp.bfloat16)


def _k_attn_finalize(offs_ref, x_ref, e_ref, k_ref, v_ref, wq, bq, wo, bo,
                     fw1, fb1, fw2, fb2, gw1m, gw1t3, gb1, gw2, gb2, o_ref):
    b = pl.program_id(0)
    j = pl.program_id(1)
    bt = x_ref.shape[1]
    o3 = offs_ref[b, 2]
    o_ref[0] = e_ref[0]

    @pl.when(j * bt + bt > o3)
    def _():
        x = x_ref[0]
        q = (_dot(x, wq[...]) + bq[...]) * (1.0 / math.sqrt(HD))
        outs = []
        for h in range(NH):
            qh = q[:, h * HD:(h + 1) * HD].astype(jnp.bfloat16)
            kh = k_ref[0][:, h * HD:(h + 1) * HD]
            vh = v_ref[0][:, h * HD:(h + 1) * HD]
            s = lax.dot_general(qh, kh, (((1,), (1,)), ((), ())),
                                preferred_element_type=jnp.float32)
            p = jnp.exp(s)
            r = 1.0 / jnp.sum(p, axis=-1, keepdims=True)
            outs.append(_dot(p.astype(jnp.bfloat16), vh) * r)
        ao = jnp.concatenate(outs, axis=1)
        y = _dot(ao, wo[...]) + bo[...]
        h1 = _gelu(_dot(y, fw1[...]) + fb1[...])
        e3 = _dot(h1, fw2[...]) + fb2[...]
        hg = _gelu(_dot(e3, gw1m[...]) + gw1t3[...] + gb1[...])
        logit = _dot(hg, gw2[...]) + gb2[...]
        g3 = e3 * jax.nn.sigmoid(logit[:, 0:1])
        rid = j * bt + lax.broadcasted_iota(jnp.int32, (bt, 1), 0)
        o_ref[0] = jnp.where(rid >= o3, g3, e_ref[0])


def _row2d(v):
    return v.reshape(1, -1)


def _wspec(shape):
    return pl.BlockSpec(shape, lambda b, j, offs: tuple(0 for _ in shape))


def _bspec(shape, bi, bj):
    return pl.BlockSpec(shape, lambda b, j, offs: (bi, bj))


def _rowspec(B, S):
    return pl.BlockSpec((1, BT, D), lambda b, j, offs: (b, j, 0))


def kernel(token_embeddings, attention_weights, params):
    te, aw, p = token_embeddings, attention_weights, params
    B, S, _ = te.shape
    nblk = S // BT

    token_attention = aw.sum(axis=-1)
    max_att = jnp.max(token_attention, axis=-1, keepdims=True)
    na = token_attention / (max_att + 1e-8)
    t_min = jax.nn.sigmoid(p["th_minimal"])
    t_std = jax.nn.sigmoid(p["th_standard"])
    t_enh = jax.nn.sigmoid(p["th_enhanced"])
    tiers = jnp.where(na >= t_enh, 3,
                      jnp.where(na >= t_std, 2, jnp.where(na >= t_min, 1, 0)))
    masks = (tiers[:, :, None] == jnp.arange(4)[None, None, :]).astype(jnp.int32)
    counts = masks.sum(axis=1)
    tier_off = jnp.cumsum(counts, axis=1) - counts
    ranks = jnp.cumsum(masks, axis=1) - masks
    inv_pos = jnp.sum(masks * (ranks + tier_off[:, None, :]), axis=2)
    o1 = counts[:, 0]
    o2 = o1 + counts[:, 1]
    o3 = o2 + counts[:, 2]
    offs = jnp.stack([o1, o2, o3, jnp.full_like(o1, S)], axis=1)

    batch_base = (jnp.arange(B, dtype=jnp.int32) * S)[:, None]
    pos_idx = (inv_pos.astype(jnp.int32) + batch_base).reshape(-1)

    xs = _sc_scatter_rows(te.reshape(B * S, D), pos_idx).reshape(B, S, D)

    grid = (B, nblk)
    row = _rowspec(B, S)
    fullrow = pl.BlockSpec((1, S, D), lambda b, j, offs: (b, 0, 0))

    def call(body, in_specs, out_shape, out_specs, args, bt=BT):
        return pl.pallas_call(
            body,
            grid_spec=pltpu.PrefetchScalarGridSpec(
                num_scalar_prefetch=1, grid=(B, S // bt),
                in_specs=in_specs, out_specs=out_specs),
            out_shape=out_shape,
            compiler_params=pltpu.CompilerParams(
                dimension_semantics=("parallel", "parallel")),
        )(offs, *args)

    f32 = jnp.float32
    bf16 = jnp.bfloat16
    shp = lambda dt=f32: jax.ShapeDtypeStruct((B, S, D), dt)
    w = lambda a: a

    w1m = w(p["cc_W1"][:D])
    w1t = w(jnp.pad(p["cc_W1"][D:], ((0, 4), (0, 0))))
    w2m = w(jnp.pad(p["cc_W2"], ((0, 0), (0, 127))))
    b1c = _row2d(p["cc_b1"])
    b2c = jnp.pad(_row2d(p["cc_b2"]), ((0, 0), (0, 127)))

    e_args = [
        xs,
        w(p["min_W1"]), _row2d(p["min_b1"]), w(p["min_W2"]), _row2d(p["min_b2"]),
        w(p["std_W1"]), _row2d(p["std_b1"]), w(p["std_W2"]), _row2d(p["std_b2"]),
        w(p["enh_a_W1"]), _row2d(p["enh_a_b1"]), w(p["enh_a_W2"]), _row2d(p["enh_a_b2"]),
        w(p["enh_b_W1"]), _row2d(p["enh_b_b1"]), w(p["enh_b_W2"]), _row2d(p["enh_b_b2"]),
        w(p["enh_b_W3"]), _row2d(p["enh_b_b3"]),
        p["enh_c_W1"], p["enh_c_W1"], _row2d(p["enh_c_b1"]),
        w(p["enh_c_W2"]), _row2d(p["enh_c_b2"]),
        w1m, w1t, b1c, w2m, b2c,
    ]
    e_specs = [row] + [_wspec(a.shape) for a in e_args[1:]]
    e_specs[19] = _bspec((D, D), 0, 0)
    e_specs[20] = _bspec((D, D), 1, 0)
    out012 = call(_k_experts012, e_specs, shp(), row, e_args)

    bq = _row2d(p["attn_bqkv"][:D])
    bk = _row2d(p["attn_bqkv"][D:2 * D])
    bv = _row2d(p["attn_bqkv"][2 * D:])

    p2_args = [xs,
               w(p["prem_0_W1"]), _row2d(p["prem_0_b1"]),
               w(p["prem_0_W2"]), _row2d(p["prem_0_b2"]),
               w(p["prem_1_W1"]), _row2d(p["prem_1_b1"]),
               w(p["prem_1_W2"]), _row2d(p["prem_1_b2"])]
    bt_trunk = 512
    row_t = pl.BlockSpec((1, bt_trunk, D), lambda b, j, offs: (b, j, 0))
    p2_specs = [row_t] + [_wspec(a.shape) for a in p2_args[1:]]
    x3 = call(_k_ffn_res2, p2_specs, shp(), row_t, p2_args, bt=bt_trunk)

    kv_args = [x3, w(p["prem_2_W1"]), _row2d(p["prem_2_b1"]),
               w(p["prem_2_W2"]), _row2d(p["prem_2_b2"]),
               p["attn_Wqkv"], bk, p["attn_Wqkv"], bv]
    kv_specs = [row_t] + [_wspec(a.shape) for a in kv_args[1:]]
    kv_specs[5] = _bspec((D, D), 0, 1)
    kv_specs[7] = _bspec((D, D), 0, 2)
    x3, karr, varr = call(_k_ffn_res_kv, kv_specs, (shp(), shp(bf16), shp(bf16)),
                          (row_t, row_t, row_t), kv_args, bt=bt_trunk)

    a_args = [x3, out012, karr, varr, p["attn_Wqkv"], bq,
              w(p["attn_Wo"]), _row2d(p["attn_bo"]),
              w(p["fin_W1"]), _row2d(p["fin_b1"]), w(p["fin_W2"]), _row2d(p["fin_b2"]),
              w1m, w1t[3:4], b1c, w2m, b2c]
    bt_f = 512
    row_f = pl.BlockSpec((1, bt_f, D), lambda b, j, offs: (b, j, 0))
    a_specs = [row_f, row_f, fullrow, fullrow] + [_wspec(a.shape) for a in a_args[4:]]
    a_specs[4] = _bspec((D, D), 0, 0)
    ys = call(_k_attn_finalize, a_specs, shp(), row_f, a_args, bt=bt_f)

    out = _sc_gather_rows(ys.reshape(B * S, D), pos_idx)
    return out.reshape(B, S, D)


# speedup vs baseline: 1.0117x; 1.0117x over previous
import functools
import math

import jax
import jax.numpy as jnp
from jax import lax
from jax.experimental import pallas as pl
from jax.experimental.pallas import tpu as pltpu
from jax.experimental.pallas import tpu_sc as plsc

D = 1024
D2 = 512
NH = 8
HD = D // NH
BT = 256


def _gelu(x):
    return 0.5 * x * (1.0 + lax.erf(x * (1.0 / math.sqrt(2.0))))


def _dot(a, b):
    return jnp.dot(a, b, preferred_element_type=jnp.float32)


_SC_CHUNK = 32


def _sc_gather_rows(table, idx):
    n_rows = idx.shape[0]
    info = plsc.get_sparse_core_info()
    num_workers = info.num_cores * info.num_subcores
    rows_per_w = n_rows // num_workers
    chunk = _SC_CHUNK
    n_chunks = rows_per_w // chunk
    mesh = plsc.VectorSubcoreMesh(core_axis_name="c", subcore_axis_name="s")

    @functools.partial(
        pl.kernel,
        mesh=mesh,
        out_type=jax.ShapeDtypeStruct((n_rows, D), jnp.float32),
        scratch_types=[
            pltpu.VMEM((2, chunk), jnp.int32),
            pltpu.VMEM((2, chunk, D), jnp.float32),
            pltpu.SemaphoreType.DMA,
            pltpu.SemaphoreType.DMA,
        ],
    )
    def k(table_hbm, idx_hbm, out_hbm, idx_v, rows_v, gsem, ssem):
        wid = lax.axis_index("s") * info.num_cores + lax.axis_index("c")
        base = wid * rows_per_w
        gh = [None] * n_chunks
        sh = [None] * n_chunks
        for c in range(n_chunks):
            buf = c % 2
            if c >= 2:
                sh[c - 2].wait()
            pltpu.sync_copy(idx_hbm.at[pl.ds(base + c * chunk, chunk)],
                            idx_v.at[buf])
            gh[c] = pltpu.async_copy(table_hbm.at[idx_v.at[buf]],
                                     rows_v.at[buf], gsem)
            if c >= 1:
                gh[c - 1].wait()
                sh[c - 1] = pltpu.async_copy(
                    rows_v.at[1 - buf],
                    out_hbm.at[pl.ds(base + (c - 1) * chunk, chunk)], ssem)
        gh[n_chunks - 1].wait()
        sh[n_chunks - 1] = pltpu.async_copy(
            rows_v.at[(n_chunks - 1) % 2],
            out_hbm.at[pl.ds(base + (n_chunks - 1) * chunk, chunk)], ssem)
        sh[n_chunks - 2].wait()
        sh[n_chunks - 1].wait()

    return k(table, idx)


def _sc_scatter_rows(table, idx):
    n_rows = idx.shape[0]
    info = plsc.get_sparse_core_info()
    num_workers = info.num_cores * info.num_subcores
    rows_per_w = n_rows // num_workers
    chunk = _SC_CHUNK
    n_chunks = rows_per_w // chunk
    mesh = plsc.VectorSubcoreMesh(core_axis_name="c", subcore_axis_name="s")

    @functools.partial(
        pl.kernel,
        mesh=mesh,
        out_type=jax.ShapeDtypeStruct((n_rows, D), jnp.float32),
        scratch_types=[
            pltpu.VMEM((2, chunk), jnp.int32),
            pltpu.VMEM((2, chunk, D), jnp.float32),
            pltpu.SemaphoreType.DMA,
        ],
    )
    def k(table_hbm, idx_hbm, out_hbm, idx_v, rows_v, wsem):
        wid = lax.axis_index("s") * info.num_cores + lax.axis_index("c")
        base = wid * rows_per_w
        wh = [None] * n_chunks
        for c in range(n_chunks):
            buf = c % 2
            if c >= 2:
                wh[c - 2].wait()
            off = base + c * chunk
            pltpu.sync_copy(idx_hbm.at[pl.ds(off, chunk)], idx_v.at[buf])
            pltpu.sync_copy(table_hbm.at[pl.ds(off, chunk)], rows_v.at[buf])
            wh[c] = pltpu.async_copy(rows_v.at[buf], out_hbm.at[idx_v.at[buf]],
                                     wsem)
        wh[n_chunks - 2].wait()
        wh[n_chunks - 1].wait()

    return k(table, idx)


def _k_experts012(offs_ref, x_ref,
                  mw1, mb1, mw2, mb2,
                  sw1, sb1, sw2, sb2,
                  aw1, ab1, aw2, ab2,
                  bw1, bb1, bw2, bb2, bw3, bb3,
                  cw1a, cw1b, cb1, cw2, cb2,
                  gw1m, gw1t, gb1, gw2, gb2,
                  o_ref):
    b = pl.program_id(0)
    j = pl.program_id(1)
    bt = x_ref.shape[1]
    row0 = j * bt
    o1 = offs_ref[b, 0]
    o2 = offs_ref[b, 1]
    o3 = offs_ref[b, 2]
    rid = row0 + lax.broadcasted_iota(jnp.int32, (bt, 1), 0)
    x = x_ref[0]

    @pl.when(row0 < o1)
    def _():
        h = jnp.maximum(_dot(x, mw1[...]) + mb1[...], 0.0)
        e0 = _dot(h, mw2[...]) + mb2[...]
        o_ref[0] = jnp.where(rid < o1, e0, o_ref[0])

    @pl.when((row0 < o2) & (row0 + bt > o1))
    def _():
        h = _gelu(_dot(x, sw1[...]) + sb1[...])
        e1 = _dot(h, sw2[...]) + sb2[...]
        o_ref[0] = jnp.where((rid >= o1) & (rid < o2), e1, o_ref[0])

    @pl.when((row0 < o3) & (row0 + bt > o2))
    def _():
        a = _dot(_gelu(_dot(x, aw1[...]) + ab1[...]), aw2[...]) + ab2[...]
        t = _gelu(_dot(x, bw1[...]) + bb1[...])
        t = _gelu(_dot(t, bw2[...]) + bb2[...])
        bb = _dot(t, bw3[...]) + bb3[...]
        h = _gelu(_dot(a, cw1a[...]) + _dot(bb, cw1b[...]) + cb1[...])
        e2 = _dot(h, cw2[...]) + cb2[...]
        o_ref[0] = jnp.where((rid >= o2) & (rid < o3), e2, o_ref[0])

    @pl.when(row0 < o3)
    def _():
        pr = o_ref[0]
        tier = ((rid >= o1).astype(jnp.int32) + (rid >= o2).astype(jnp.int32)
                + (rid >= o3).astype(jnp.int32))
        cols = lax.broadcasted_iota(jnp.int32, (bt, 8), 1)
        onehot = (tier == cols).astype(jnp.float32)
        h = _gelu(_dot(pr, gw1m[...]) + _dot(onehot, gw1t[...]) + gb1[...])
        logit = _dot(h, gw2[...]) + gb2[...]
        o_ref[0] = pr * jax.nn.sigmoid(logit[:, 0:1])


def _k_ffn_res2(offs_ref, x_ref, w1a, b1a, w2a, b2a, w1b, b1b, w2b, b2b, o_ref):
    b = pl.program_id(0)

    @pl.when(offs_ref[b, 3] > offs_ref[b, 2])
    def _():
        x = x_ref[0]
        h = _gelu(_dot(x, w1a[...]) + b1a[...])
        x = x + _dot(h, w2a[...]) + b2a[...]
        h = _gelu(_dot(x, w1b[...]) + b1b[...])
        o_ref[0] = x + _dot(h, w2b[...]) + b2b[...]


def _k_ffn_res_kv(offs_ref, x_ref, w1, b1, w2, b2, wk, bk, wv, bv,
                  o_ref, k_ref, v_ref):
    b = pl.program_id(0)

    @pl.when(offs_ref[b, 3] > offs_ref[b, 2])
    def _():
        x = x_ref[0]
        h = _gelu(_dot(x, w1[...]) + b1[...])
        x3 = x + _dot(h, w2[...]) + b2[...]
        o_ref[0] = x3
        k_ref[0] = (_dot(x3, wk[...]) + bk[...]).astype(jnp.bfloat16)
        v_ref[0] = (_dot(x3, wv[...]) + bv[...]).astype(jnp.bfloat16)


def _k_attn_finalize(offs_ref, x_ref, e_ref, k_ref, v_ref, wq, bq, wo, bo,
                     fw1, fb1, fw2, fb2, gw1m, gw1t3, gb1, gw2, gb2, o_ref):
    b = pl.program_id(0)
    j = pl.program_id(1)
    bt = x_ref.shape[1]
    o3 = offs_ref[b, 2]
    o_ref[0] = e_ref[0]

    @pl.when(j * bt + bt > o3)
    def _():
        x = x_ref[0]
        q = (_dot(x, wq[...]) + bq[...]) * (1.0 / math.sqrt(HD))
        outs = []
        for h in range(NH):
            qh = q[:, h * HD:(h + 1) * HD].astype(jnp.bfloat16)
            kh = k_ref[0][:, h * HD:(h + 1) * HD]
            vh = v_ref[0][:, h * HD:(h + 1) * HD]
            s = lax.dot_general(qh, kh, (((1,), (1,)), ((), ())),
                                preferred_element_type=jnp.float32)
            p = jnp.exp(s)
            r = 1.0 / jnp.sum(p, axis=-1, keepdims=True)
            outs.append(_dot(p.astype(jnp.bfloat16), vh) * r)
        ao = jnp.concatenate(outs, axis=1)
        y = _dot(ao, wo[...]) + bo[...]
        h1 = _gelu(_dot(y, fw1[...]) + fb1[...])
        e3 = _dot(h1, fw2[...]) + fb2[...]
        hg = _gelu(_dot(e3, gw1m[...]) + gw1t3[...] + gb1[...])
        logit = _dot(hg, gw2[...]) + gb2[...]
        g3 = e3 * jax.nn.sigmoid(logit[:, 0:1])
        rid = j * bt + lax.broadcasted_iota(jnp.int32, (bt, 1), 0)
        o_ref[0] = jnp.where(rid >= o3, g3, e_ref[0])


def _row2d(v):
    return v.reshape(1, -1)


def _wspec(shape):
    return pl.BlockSpec(shape, lambda b, j, offs: tuple(0 for _ in shape))


def _bspec(shape, bi, bj):
    return pl.BlockSpec(shape, lambda b, j, offs: (bi, bj))


def _rowspec(B, S):
    return pl.BlockSpec((1, BT, D), lambda b, j, offs: (b, j, 0))


def kernel(token_embeddings, attention_weights, params):
    te, aw, p = token_embeddings, attention_weights, params
    B, S, _ = te.shape
    nblk = S // BT

    token_attention = aw.sum(axis=-1)
    max_att = jnp.max(token_attention, axis=-1, keepdims=True)
    na = token_attention / (max_att + 1e-8)
    t_min = jax.nn.sigmoid(p["th_minimal"])
    t_std = jax.nn.sigmoid(p["th_standard"])
    t_enh = jax.nn.sigmoid(p["th_enhanced"])
    tiers = jnp.where(na >= t_enh, 3,
                      jnp.where(na >= t_std, 2, jnp.where(na >= t_min, 1, 0)))
    masks = (tiers[:, :, None] == jnp.arange(4)[None, None, :]).astype(jnp.int32)
    counts = masks.sum(axis=1)
    tier_off = jnp.cumsum(counts, axis=1) - counts
    ranks = jnp.cumsum(masks, axis=1) - masks
    inv_pos = jnp.sum(masks * (ranks + tier_off[:, None, :]), axis=2)
    o1 = counts[:, 0]
    o2 = o1 + counts[:, 1]
    o3 = o2 + counts[:, 2]
    offs = jnp.stack([o1, o2, o3, jnp.full_like(o1, S)], axis=1)

    batch_base = (jnp.arange(B, dtype=jnp.int32) * S)[:, None]
    pos_idx = (inv_pos.astype(jnp.int32) + batch_base).reshape(-1)

    xs = _sc_scatter_rows(te.reshape(B * S, D), pos_idx).reshape(B, S, D)

    grid = (B, nblk)
    row = _rowspec(B, S)
    fullrow = pl.BlockSpec((1, S, D), lambda b, j, offs: (b, 0, 0))

    def call(body, in_specs, out_shape, out_specs, args, bt=BT):
        return pl.pallas_call(
            body,
            grid_spec=pltpu.PrefetchScalarGridSpec(
                num_scalar_prefetch=1, grid=(B, S // bt),
                in_specs=in_specs, out_specs=out_specs),
            out_shape=out_shape,
            compiler_params=pltpu.CompilerParams(
                dimension_semantics=("parallel", "parallel")),
        )(offs, *args)

    f32 = jnp.float32
    bf16 = jnp.bfloat16
    shp = lambda dt=f32: jax.ShapeDtypeStruct((B, S, D), dt)
    w = lambda a: a

    w1m = w(p["cc_W1"][:D])
    w1t = w(jnp.pad(p["cc_W1"][D:], ((0, 4), (0, 0))))
    w2m = w(jnp.pad(p["cc_W2"], ((0, 0), (0, 127))))
    b1c = _row2d(p["cc_b1"])
    b2c = jnp.pad(_row2d(p["cc_b2"]), ((0, 0), (0, 127)))

    e_args = [
        xs,
        w(p["min_W1"]), _row2d(p["min_b1"]), w(p["min_W2"]), _row2d(p["min_b2"]),
        w(p["std_W1"]), _row2d(p["std_b1"]), w(p["std_W2"]), _row2d(p["std_b2"]),
        w(p["enh_a_W1"]), _row2d(p["enh_a_b1"]), w(p["enh_a_W2"]), _row2d(p["enh_a_b2"]),
        w(p["enh_b_W1"]), _row2d(p["enh_b_b1"]), w(p["enh_b_W2"]), _row2d(p["enh_b_b2"]),
        w(p["enh_b_W3"]), _row2d(p["enh_b_b3"]),
        p["enh_c_W1"], p["enh_c_W1"], _row2d(p["enh_c_b1"]),
        w(p["enh_c_W2"]), _row2d(p["enh_c_b2"]),
        w1m, w1t, b1c, w2m, b2c,
    ]
    e_specs = [row] + [_wspec(a.shape) for a in e_args[1:]]
    e_specs[19] = _bspec((D, D), 0, 0)
    e_specs[20] = _bspec((D, D), 1, 0)
    out012 = call(_k_experts012, e_specs, shp(), row, e_args)

    bq = _row2d(p["attn_bqkv"][:D])
    bk = _row2d(p["attn_bqkv"][D:2 * D])
    bv = _row2d(p["attn_bqkv"][2 * D:])

    p2_args = [xs,
               w(p["prem_0_W1"]), _row2d(p["prem_0_b1"]),
               w(p["prem_0_W2"]), _row2d(p["prem_0_b2"]),
               w(p["prem_1_W1"]), _row2d(p["prem_1_b1"]),
               w(p["prem_1_W2"]), _row2d(p["prem_1_b2"])]
    bt_trunk = 512
    row_t = pl.BlockSpec((1, bt_trunk, D), lambda b, j, offs: (b, j, 0))
    p2_specs = [row_t] + [_wspec(a.shape) for a in p2_args[1:]]
    x3 = call(_k_ffn_res2, p2_specs, shp(), row_t, p2_args, bt=bt_trunk)

    kv_args = [x3, w(p["prem_2_W1"]), _row2d(p["prem_2_b1"]),
               w(p["prem_2_W2"]), _row2d(p["prem_2_b2"]),
               p["attn_Wqkv"], bk, p["attn_Wqkv"], bv]
    kv_specs = [row_t] + [_wspec(a.shape) for a in kv_args[1:]]
    kv_specs[5] = _bspec((D, D), 0, 1)
    kv_specs[7] = _bspec((D, D), 0, 2)
    x3, karr, varr = call(_k_ffn_res_kv, kv_specs, (shp(), shp(bf16), shp(bf16)),
                          (row_t, row_t, row_t), kv_args, bt=bt_trunk)

    a_args = [x3, out012, karr, varr, p["attn_Wqkv"], bq,
              w(p["attn_Wo"]), _row2d(p["attn_bo"]),
              w(p["fin_W1"]), _row2d(p["fin_b1"]), w(p["fin_W2"]), _row2d(p["fin_b2"]),
              w1m, w1t[3:4], b1c, w2m, b2c]
    a_specs = [row, row, fullrow, fullrow] + [_wspec(a.shape) for a in a_args[4:]]
    a_specs[4] = _bspec((D, D), 0, 0)
    ys = call(_k_attn_finalize, a_specs, shp(), row, a_args)

    out = _sc_gather_rows(ys.reshape(B * S, D), pos_idx)
    return out.reshape(B, S, D)
